# baseline (speedup 1.0000x reference)
import functools

import jax
import jax.numpy as jnp
from jax import lax
from jax.experimental import pallas as pl
from jax.experimental.pallas import tpu as pltpu
from jax.experimental.pallas import tpu_sc as plsc

N, D_IN, H, D_OUT, E, K = 4096, 1024, 1024, 1024, 8, 2
NK = N * K
BM = 256
T = NK // BM + E
S = T * BM
NC, NS = 2, 16
NW = NC * NS
CHUNK = NK // NW
RROWS = 64
RND = CHUNK // RROWS
DW = D_IN // 2


def _leaky(x, slope=0.01):
    return jnp.where(x >= 0, x, slope * x)


def _shift_down(x, s):
    return jnp.concatenate([jnp.zeros((s,) + x.shape[1:], x.dtype), x[:-s]], 0)


def _router_body(x_ref, rW1_ref, rb1_ref, rW2_ref, rb2_ref,
                 w_ref, ei_ref, rank_ref, cnt_ref, base_ref):
    @pl.when(pl.program_id(0) == 0)
    def _():
        base_ref[...] = jnp.zeros((1, E), jnp.int32)

    xb = x_ref[...]
    h = jnp.dot(xb, rW1_ref[...], preferred_element_type=jnp.float32)
    h = _leaky(h + rb1_ref[...])
    logits = jnp.dot(h, rW2_ref[...], preferred_element_type=jnp.float32)
    logits = logits + rb2_ref[...]

    bm = logits.shape[0]
    ids = lax.broadcasted_iota(jnp.int32, (bm, E), 1)
    m1 = jnp.max(logits, axis=1, keepdims=True)
    i1 = jnp.min(jnp.where(logits == m1, ids, E), axis=1, keepdims=True)
    masked = jnp.where(ids == i1, -jnp.inf, logits)
    m2 = jnp.max(masked, axis=1, keepdims=True)
    i2 = jnp.min(jnp.where(masked == m2, ids, E), axis=1, keepdims=True)
    e2 = jnp.exp(m2 - m1)
    p1 = 1.0 / (1.0 + e2)
    p2 = e2 / (1.0 + e2)
    w_ref[...] = jnp.concatenate([p1, p2], axis=1)
    ei_ref[...] = jnp.concatenate([i1, i2], axis=1)

    oh0 = (ids == i1).astype(jnp.int32)
    oh1 = (ids == i2).astype(jnp.int32)
    both = oh0 + oh1
    incl = both
    sft = 1
    while sft < bm:
        incl = incl + _shift_down(incl, sft)
        sft *= 2
    cum = incl - both
    base = base_ref[...]
    rank0 = jnp.sum(oh0 * (cum + base), axis=1, keepdims=True)
    rank1 = jnp.sum(oh1 * (cum + oh0 + base), axis=1, keepdims=True)
    rank_ref[...] = jnp.concatenate([rank0, rank1], axis=1)
    new_base = base + jnp.sum(both, axis=0, keepdims=True)
    base_ref[...] = new_base
    cnt_ref[...] = new_base


def _router(x, rW1, rb1, rW2, rb2):
    bm = 256
    return pl.pallas_call(
        _router_body,
        grid=(N // bm,),
        in_specs=[
            pl.BlockSpec((bm, D_IN), lambda i: (i, 0)),
            pl.BlockSpec((D_IN, H), lambda i: (0, 0)),
            pl.BlockSpec((1, H), lambda i: (0, 0)),
            pl.BlockSpec((H, E), lambda i: (0, 0)),
            pl.BlockSpec((1, E), lambda i: (0, 0)),
        ],
        out_specs=[
            pl.BlockSpec((bm, K), lambda i: (i, 0)),
            pl.BlockSpec((bm, K), lambda i: (i, 0)),
            pl.BlockSpec((bm, K), lambda i: (i, 0)),
            pl.BlockSpec((1, E), lambda i: (0, 0)),
        ],
        out_shape=[
            jax.ShapeDtypeStruct((N, K), jnp.float32),
            jax.ShapeDtypeStruct((N, K), jnp.int32),
            jax.ShapeDtypeStruct((N, K), jnp.int32),
            jax.ShapeDtypeStruct((1, E), jnp.int32),
        ],
        scratch_shapes=[pltpu.VMEM((1, E), jnp.int32)],
    )(x, rW1, rb1.reshape(1, H), rW2, rb2.reshape(1, E))


def _iota16():
    return lax.iota(jnp.int32, 16)


def _ci(v):
    return jnp.full((16,), v, jnp.int32)


def _dispatch_body(slot_hbm, x_hbm, xg_hbm, sidx_v, tok_v,
                   rows_a, rows_b, gs_a, gs_b, ss_a, ss_b):
    c = lax.axis_index("c")
    s = lax.axis_index("s")
    wid = s * NC + c
    base_i = wid * CHUNK
    iot = _iota16()

    for r in range(RND):
        pltpu.sync_copy(slot_hbm.at[pl.ds(base_i + r * RROWS, RROWS)],
                        sidx_v.at[r, 0])
    for jj in range(CHUNK // 16):
        item = jnp.broadcast_to(base_i + jj * 16, (16,)) + iot
        tok_v[pl.ds(jj * 16, 16)] = lax.shift_right_logical(item, _ci(1))

    bufs = (rows_a, rows_b)
    gsems = (gs_a, gs_b)
    ssems = (ss_a, ss_b)
    gathers = [None] * RND
    scatters = [None] * RND
    gathers[0] = pltpu.async_copy(
        x_hbm.at[tok_v.at[pl.ds(0, RROWS)]], bufs[0], gsems[0])
    for r in range(RND):
        gathers[r].wait()
        if r >= 1:
            scatters[r - 1].wait()
        if r + 1 < RND:
            gathers[r + 1] = pltpu.async_copy(
                x_hbm.at[tok_v.at[pl.ds((r + 1) * RROWS, RROWS)]],
                bufs[(r + 1) % 2], gsems[(r + 1) % 2])
        scatters[r] = pltpu.async_copy(
            bufs[r % 2], xg_hbm.at[sidx_v.at[r, 0]], ssems[r % 2])
    scatters[RND - 1].wait()


def _dispatch(slot, x_i32):
    mesh = plsc.VectorSubcoreMesh(core_axis_name="c", subcore_axis_name="s")
    fn = functools.partial(
        pl.kernel,
        mesh=mesh,
        out_type=jax.ShapeDtypeStruct((S, DW), jnp.int32),
        scratch_types=[
            pltpu.VMEM((RND, 1, RROWS), jnp.int32),
            pltpu.VMEM((CHUNK,), jnp.int32),
            pltpu.VMEM((RROWS, DW), jnp.int32),
            pltpu.VMEM((RROWS, DW), jnp.int32),
            pltpu.SemaphoreType.DMA,
            pltpu.SemaphoreType.DMA,
            pltpu.SemaphoreType.DMA,
            pltpu.SemaphoreType.DMA,
        ],
    )(_dispatch_body)
    return fn(slot, x_i32)


def _gmm_body(te_ref, xg_ref, eW_ref, eb_ref, y_ref):
    acc = jnp.dot(xg_ref[...], eW_ref[0], preferred_element_type=jnp.float32)
    y_ref[...] = (acc + eb_ref[0]).astype(jnp.bfloat16)


def _grouped_matmul(xg, eW_bf, eb, tile_expert):
    grid_spec = pltpu.PrefetchScalarGridSpec(
        num_scalar_prefetch=1,
        grid=(T,),
        in_specs=[
            pl.BlockSpec((BM, D_IN), lambda t, te: (t, 0)),
            pl.BlockSpec((1, D_IN, D_OUT), lambda t, te: (te[t], 0, 0)),
            pl.BlockSpec((1, 1, D_OUT), lambda t, te: (te[t], 0, 0)),
        ],
        out_specs=pl.BlockSpec((BM, D_OUT), lambda t, te: (t, 0)),
    )
    return pl.pallas_call(
        _gmm_body,
        grid_spec=grid_spec,
        out_shape=jax.ShapeDtypeStruct((S, D_OUT), jnp.bfloat16),
    )(tile_expert, xg, eW_bf, eb.reshape(E, 1, D_OUT))


@jax.jit
def kernel(x, rW1, rb1, rW2, rb2, eW, eb):
    w, ei, rank, cnt = _router(x, rW1, rb1, rW2, rb2)
    counts = cnt[0]
    tiles_per_e = (counts + BM - 1) // BM
    tcum = jnp.cumsum(tiles_per_e)
    slot_start = (tcum - tiles_per_e) * BM
    flat_e = ei.reshape(-1)
    slot = jnp.take(slot_start, flat_e) + rank.reshape(-1)
    tile_expert = jnp.minimum(
        jnp.searchsorted(tcum, jnp.arange(T, dtype=jnp.int32),
                         side='right').astype(jnp.int32), E - 1)
    x_i32 = lax.bitcast_convert_type(
        x.astype(jnp.bfloat16).reshape(N, DW, 2), jnp.int32)
    xg_i32 = _dispatch(slot, x_i32)
    xg = lax.bitcast_convert_type(xg_i32, jnp.bfloat16).reshape(S, D_IN)
    eW_bf = eW.astype(jnp.bfloat16)
    y = _grouped_matmul(xg, eW_bf, eb, tile_expert)
    ts = slot.reshape(N, K)
    out = _leaky(w[:, 0:1] * jnp.take(y, ts[:, 0], axis=0)
                 + w[:, 1:2] * jnp.take(y, ts[:, 1], axis=0))
    return out


# speedup vs baseline: 4.5203x; 4.5203x over previous
import functools

import jax
import jax.numpy as jnp
from jax.experimental import pallas as pl
from jax.experimental.pallas import tpu as pltpu


def _leaky(x, slope=0.01):
    return jnp.where(x >= 0, x, slope * x)


def _moe_body(x_ref, rW1_ref, rb1_ref, rW2_ref, rb2_ref, eW_ref, eb_ref,
              out_ref, *, n_exp):
    xb = x_ref[...]
    h = jnp.dot(xb, rW1_ref[...], preferred_element_type=jnp.float32)
    h = _leaky(h + rb1_ref[...])
    logits = jnp.dot(h, rW2_ref[...], preferred_element_type=jnp.float32)
    logits = logits + rb2_ref[...]

    bm = logits.shape[0]
    ids = jax.lax.broadcasted_iota(jnp.int32, (bm, n_exp), 1)
    m1 = jnp.max(logits, axis=1, keepdims=True)
    i1 = jnp.min(jnp.where(logits == m1, ids, n_exp), axis=1, keepdims=True)
    masked = jnp.where(ids == i1, -jnp.inf, logits)
    m2 = jnp.max(masked, axis=1, keepdims=True)
    i2 = jnp.min(jnp.where(masked == m2, ids, n_exp), axis=1, keepdims=True)
    e2 = jnp.exp(m2 - m1)
    p1 = 1.0 / (1.0 + e2)
    p2 = e2 / (1.0 + e2)
    coef = jnp.where(ids == i1, p1, 0.0) + jnp.where(ids == i2, p2, 0.0)

    acc = jnp.dot(coef, eb_ref[...], preferred_element_type=jnp.float32)
    xbf = xb.astype(jnp.bfloat16)
    for e in range(n_exp):
        y = jnp.dot(xbf, eW_ref[e], preferred_element_type=jnp.float32)
        acc = acc + coef[:, e:e + 1] * y
    out_ref[...] = _leaky(acc)


@jax.jit
def kernel(x, rW1, rb1, rW2, rb2, eW, eb):
    n, d_in = x.shape
    h_dim = rW1.shape[1]
    n_exp = eW.shape[0]
    d_out = eW.shape[2]
    bm = min(512, n)
    grid = (n // bm,)

    eW_bf = eW.astype(jnp.bfloat16)

    out = pl.pallas_call(
        functools.partial(_moe_body, n_exp=n_exp),
        grid=grid,
        in_specs=[
            pl.BlockSpec((bm, d_in), lambda i: (i, 0)),
            pl.BlockSpec((d_in, h_dim), lambda i: (0, 0)),
            pl.BlockSpec((1, h_dim), lambda i: (0, 0)),
            pl.BlockSpec((h_dim, n_exp), lambda i: (0, 0)),
            pl.BlockSpec((1, n_exp), lambda i: (0, 0)),
            pl.BlockSpec((n_exp, d_in, d_out), lambda i: (0, 0, 0)),
            pl.BlockSpec((n_exp, d_out), lambda i: (0, 0)),
        ],
        out_specs=pl.BlockSpec((bm, d_out), lambda i: (i, 0)),
        out_shape=jax.ShapeDtypeStruct((n, d_out), jnp.float32),
        compiler_params=pltpu.CompilerParams(
            dimension_semantics=("arbitrary",)),
    )(x, rW1, rb1.reshape(1, h_dim), rW2, rb2.reshape(1, n_exp), eW_bf, eb)
    return out


# speedup vs baseline: 4.5277x; 1.0016x over previous
import functools

import jax
import jax.numpy as jnp
from jax.experimental import pallas as pl
from jax.experimental.pallas import tpu as pltpu


def _leaky(x, slope=0.01):
    return jnp.where(x >= 0, x, slope * x)


def _moe_body(x_ref, rW1_ref, rb1_ref, rW2_ref, rb2_ref, eW_ref, eb_ref,
              out_ref, *, n_exp):
    xb = x_ref[...]
    h = jnp.dot(xb, rW1_ref[...], preferred_element_type=jnp.float32)
    h = _leaky(h + rb1_ref[...])
    logits = jnp.dot(h, rW2_ref[...], preferred_element_type=jnp.float32)
    logits = logits + rb2_ref[...]

    bm = logits.shape[0]
    ids = jax.lax.broadcasted_iota(jnp.int32, (bm, n_exp), 1)
    m1 = jnp.max(logits, axis=1, keepdims=True)
    i1 = jnp.min(jnp.where(logits == m1, ids, n_exp), axis=1, keepdims=True)
    masked = jnp.where(ids == i1, -jnp.inf, logits)
    m2 = jnp.max(masked, axis=1, keepdims=True)
    i2 = jnp.min(jnp.where(masked == m2, ids, n_exp), axis=1, keepdims=True)
    e2 = jnp.exp(m2 - m1)
    p1 = 1.0 / (1.0 + e2)
    p2 = e2 / (1.0 + e2)
    coef = jnp.where(ids == i1, p1, 0.0) + jnp.where(ids == i2, p2, 0.0)

    acc = jnp.dot(coef, eb_ref[...], preferred_element_type=jnp.float32)
    xbf = xb.astype(jnp.bfloat16)
    for e in range(n_exp):
        y = jnp.dot(xbf, eW_ref[e], preferred_element_type=jnp.float32)
        acc = acc + coef[:, e:e + 1] * y
    out_ref[...] = _leaky(acc)


@jax.jit
def kernel(x, rW1, rb1, rW2, rb2, eW, eb):
    n, d_in = x.shape
    h_dim = rW1.shape[1]
    n_exp = eW.shape[0]
    d_out = eW.shape[2]
    bm = min(1024, n)
    grid = (n // bm,)

    eW_bf = eW.astype(jnp.bfloat16)

    out = pl.pallas_call(
        functools.partial(_moe_body, n_exp=n_exp),
        grid=grid,
        in_specs=[
            pl.BlockSpec((bm, d_in), lambda i: (i, 0)),
            pl.BlockSpec((d_in, h_dim), lambda i: (0, 0)),
            pl.BlockSpec((1, h_dim), lambda i: (0, 0)),
            pl.BlockSpec((h_dim, n_exp), lambda i: (0, 0)),
            pl.BlockSpec((1, n_exp), lambda i: (0, 0)),
            pl.BlockSpec((n_exp, d_in, d_out), lambda i: (0, 0, 0)),
            pl.BlockSpec((n_exp, d_out), lambda i: (0, 0)),
        ],
        out_specs=pl.BlockSpec((bm, d_out), lambda i: (i, 0)),
        out_shape=jax.ShapeDtypeStruct((n, d_out), jnp.float32),
        compiler_params=pltpu.CompilerParams(
            dimension_semantics=("arbitrary",)),
    )(x, rW1, rb1.reshape(1, h_dim), rW2, rb2.reshape(1, n_exp), eW_bf, eb)
    return out
